# baseline (speedup 1.0000x reference)
import functools

import jax
import jax.numpy as jnp
from jax import lax
from jax.experimental import pallas as pl
from jax.experimental.pallas import tpu as pltpu
from jax.experimental.pallas import tpu_sc as plsc

BATCH = 16384
EMBED_DIM = 64
NROWS = 1000000


@functools.lru_cache(maxsize=None)
def _make_transpose(nrows, dim):
    info = plsc.get_sparse_core_info()
    nc, ns = info.num_cores, info.num_subcores
    nw = nc * ns
    nfull = nrows // 128
    tail = nrows - nfull * 128
    kmax = -(-nfull // nw)
    tail_wid = nfull % nw
    mesh = plsc.VectorSubcoreMesh(core_axis_name="c", subcore_axis_name="s")

    @functools.partial(
        pl.kernel,
        mesh=mesh,
        out_type=jax.ShapeDtypeStruct((nrows, dim), jnp.float32),
        scratch_types=[
            pltpu.VMEM((dim, 128), jnp.float32),
            pltpu.VMEM((dim, tail), jnp.float32),
            pltpu.VMEM((128, dim), jnp.float32),
            pltpu.SemaphoreType.DMA,
        ],
        compiler_params=pltpu.CompilerParams(needs_layout_passes=False),
    )
    def tkern(tt_hbm, out_hbm, slab_v, tail_v, tslab_v, sem):
        wid = lax.axis_index("s") * nc + lax.axis_index("c")
        lanes = lax.iota(jnp.int32, 16)
        jvecs = [lanes + (16 * k) for k in range(dim // 16)]
        zeros = jnp.full((16,), 0, jnp.int32)

        def transpose_rows(src, nr):
            for i in range(nr):
                ivec = zeros + i
                for k in range(dim // 16):
                    tslab_v[i, pl.ds(16 * k, 16)] = plsc.load_gather(
                        src, [jvecs[k], ivec])

        def body(k, _):
            s = wid + nw * k

            @pl.when(s < nfull)
            def _():
                c0 = s * 128
                pltpu.async_copy(
                    tt_hbm.at[:, pl.ds(c0, 128)], slab_v, sem).wait()
                transpose_rows(slab_v, 128)
                pltpu.sync_copy(tslab_v, out_hbm.at[pl.ds(c0, 128)])

            return 0

        lax.fori_loop(0, kmax, body, 0)

        if tail:
            @pl.when(wid == tail_wid)
            def _():
                c0 = nfull * 128
                pltpu.async_copy(
                    tt_hbm.at[:, pl.ds(c0, tail)], tail_v, sem).wait()
                transpose_rows(tail_v, tail)
                pltpu.sync_copy(tslab_v.at[pl.ds(0, tail)],
                                out_hbm.at[pl.ds(c0, tail)])

    return tkern


@functools.lru_cache(maxsize=None)
def _make_gather(batch, dim):
    info = plsc.get_sparse_core_info()
    nc, ns = info.num_cores, info.num_subcores
    nw = nc * ns
    assert batch % (8 * nw) == 0
    bpw = batch // nw
    mesh = plsc.VectorSubcoreMesh(core_axis_name="c", subcore_axis_name="s")

    @functools.partial(
        pl.kernel,
        mesh=mesh,
        out_type=(
            jax.ShapeDtypeStruct((batch, dim), jnp.float32),
            jax.ShapeDtypeStruct((batch, dim), jnp.float32),
        ),
        scratch_types=[
            pltpu.VMEM((bpw,), jnp.int32),
            pltpu.VMEM((bpw,), jnp.int32),
            pltpu.VMEM((bpw, dim), jnp.float32),
            pltpu.SemaphoreType.DMA,
        ],
    )
    def gather2(ut_hbm, uid_hbm, it_hbm, iid_hbm, uo_hbm, io_hbm,
                uidx_v, iidx_v, rows_v, sem):
        wid = lax.axis_index("s") * nc + lax.axis_index("c")
        base = wid * bpw
        pltpu.sync_copy(uid_hbm.at[pl.ds(base, bpw)], uidx_v)
        pltpu.sync_copy(iid_hbm.at[pl.ds(base, bpw)], iidx_v)

        def one_table(tab, idx_v, out_hbm):
            def fire(g, _):
                vec = idx_v[pl.ds(g * 16, 16)]
                for lane in range(16):
                    rid = lax.squeeze(
                        lax.slice(vec, (lane,), (lane + 1,)), (0,))
                    pltpu.async_copy(tab.at[pl.ds(rid, 1)],
                                     rows_v.at[pl.ds(g * 16 + lane, 1)], sem)
                return 0

            lax.fori_loop(0, bpw // 16, fire, 0)

            def drain(i, _):
                pltpu.make_async_copy(tab.at[pl.ds(0, 1)],
                                      rows_v.at[pl.ds(0, 1)], sem).wait()
                return 0

            lax.fori_loop(0, bpw, drain, 0)
            pltpu.sync_copy(rows_v, out_hbm.at[pl.ds(base, bpw)])

        one_table(ut_hbm, uidx_v, uo_hbm)
        one_table(it_hbm, iidx_v, io_hbm)

    return gather2


_BLK = 2048


def _mlp_body(u_ref, v_ref, w0u_ref, w0i_ref, b0_ref, w1_ref, b1_ref,
              w2_ref, b2_ref, wo_ref, bo_ref, out_ref):
    dot = functools.partial(jnp.dot, preferred_element_type=jnp.float32)
    h = dot(u_ref[...], w0u_ref[...]) + dot(v_ref[...], w0i_ref[...])
    h = jnp.maximum(h + b0_ref[...], 0.0)
    h = jnp.maximum(dot(h, w1_ref[...]) + b1_ref[...], 0.0)
    h = jnp.maximum(dot(h, w2_ref[...]) + b2_ref[...], 0.0)
    o = dot(h, wo_ref[...]) + bo_ref[...]
    out_ref[...] = jax.nn.sigmoid(o)


def _mlp(u, v, w0u, w0i, b0, w1t, b1, w2t, b2, wot, bo):
    grid = BATCH // _BLK
    row = lambda i: (i, 0)
    rep = lambda i: (0, 0)
    return pl.pallas_call(
        _mlp_body,
        grid=(grid,),
        in_specs=[
            pl.BlockSpec((_BLK, EMBED_DIM), row),
            pl.BlockSpec((_BLK, EMBED_DIM), row),
            pl.BlockSpec(w0u.shape, rep),
            pl.BlockSpec(w0i.shape, rep),
            pl.BlockSpec(b0.shape, rep),
            pl.BlockSpec(w1t.shape, rep),
            pl.BlockSpec(b1.shape, rep),
            pl.BlockSpec(w2t.shape, rep),
            pl.BlockSpec(b2.shape, rep),
            pl.BlockSpec(wot.shape, rep),
            pl.BlockSpec(bo.shape, rep),
        ],
        out_specs=pl.BlockSpec((_BLK, 1), row),
        out_shape=jax.ShapeDtypeStruct((BATCH, 1), jnp.float32),
    )(u, v, w0u, w0i, b0, w1t, b1, w2t, b2, wot, bo)


def kernel(user_ids, item_ids, user_table, item_table,
           W0, b0, W1, b1, W2, b2, Wo, bo):
    item_rm = _make_transpose(NROWS, EMBED_DIM)(item_table.T)
    u_emb, i_emb = _make_gather(BATCH, EMBED_DIM)(
        user_table, user_ids.astype(jnp.int32),
        item_rm, item_ids.astype(jnp.int32))
    w0u = W0[:, :EMBED_DIM].T
    w0i = W0[:, EMBED_DIM:].T
    return _mlp(u_emb, i_emb, w0u, w0i, b0.reshape(1, -1),
                W1.T, b1.reshape(1, -1), W2.T, b2.reshape(1, -1),
                Wo.T, bo.reshape(1, 1))


# speedup vs baseline: 2.0585x; 2.0585x over previous
import functools

import jax
import jax.numpy as jnp
from jax import lax
from jax.experimental import pallas as pl
from jax.experimental.pallas import tpu as pltpu
from jax.experimental.pallas import tpu_sc as plsc

BATCH = 16384
EMBED_DIM = 64


@functools.lru_cache(maxsize=None)
def _make_gather_stream(batch, dim):
    info = plsc.get_sparse_core_info()
    nc, ns = info.num_cores, info.num_subcores
    nw = nc * ns
    assert batch % (8 * nw) == 0
    bpw = batch // nw
    mesh = plsc.VectorSubcoreMesh(core_axis_name="c", subcore_axis_name="s")

    @functools.partial(
        pl.kernel,
        mesh=mesh,
        out_type=jax.ShapeDtypeStruct((batch, dim), jnp.float32),
        scratch_types=[
            pltpu.VMEM((bpw,), jnp.int32),
            pltpu.VMEM((bpw, dim), jnp.float32),
            pltpu.SemaphoreType.DMA,
        ],
        compiler_params=pltpu.CompilerParams(use_tc_tiling_on_sc=False),
    )
    def gather_stream(tab_hbm, id_hbm, out_hbm, idx_v, rows_v, sem):
        wid = lax.axis_index("s") * nc + lax.axis_index("c")
        base = wid * bpw
        pltpu.sync_copy(id_hbm.at[pl.ds(base, bpw)], idx_v)
        pltpu.async_copy(tab_hbm.at[idx_v], rows_v, sem).wait()
        pltpu.sync_copy(rows_v, out_hbm.at[pl.ds(base, bpw)])

    return gather_stream


@functools.lru_cache(maxsize=None)
def _make_gather_dma(batch, dim):
    info = plsc.get_sparse_core_info()
    nc, ns = info.num_cores, info.num_subcores
    nw = nc * ns
    assert batch % (8 * nw) == 0
    bpw = batch // nw
    mesh = plsc.VectorSubcoreMesh(core_axis_name="c", subcore_axis_name="s")

    @functools.partial(
        pl.kernel,
        mesh=mesh,
        out_type=jax.ShapeDtypeStruct((batch, dim), jnp.float32),
        scratch_types=[
            pltpu.VMEM((bpw,), jnp.int32),
            pltpu.VMEM((bpw, dim), jnp.float32),
            pltpu.SemaphoreType.DMA,
        ],
    )
    def gather_dma(tab_hbm, id_hbm, out_hbm, idx_v, rows_v, sem):
        wid = lax.axis_index("s") * nc + lax.axis_index("c")
        base = wid * bpw
        pltpu.sync_copy(id_hbm.at[pl.ds(base, bpw)], idx_v)

        def fire(g, _):
            vec = idx_v[pl.ds(g * 16, 16)]
            for lane in range(16):
                rid = lax.squeeze(lax.slice(vec, (lane,), (lane + 1,)), (0,))
                pltpu.async_copy(tab_hbm.at[pl.ds(rid, 1)],
                                 rows_v.at[pl.ds(g * 16 + lane, 1)], sem)
            return 0

        lax.fori_loop(0, bpw // 16, fire, 0)

        def drain(i, _):
            pltpu.make_async_copy(tab_hbm.at[pl.ds(0, 1)],
                                  rows_v.at[pl.ds(0, 1)], sem).wait()
            return 0

        lax.fori_loop(0, bpw, drain, 0)
        pltpu.sync_copy(rows_v, out_hbm.at[pl.ds(base, bpw)])

    return gather_dma


_BLK = 2048


def _mlp_body(u_ref, v_ref, w0u_ref, w0i_ref, b0_ref, w1_ref, b1_ref,
              w2_ref, b2_ref, wo_ref, bo_ref, out_ref):
    dot = functools.partial(jnp.dot, preferred_element_type=jnp.float32)
    h = dot(u_ref[...], w0u_ref[...]) + dot(v_ref[...], w0i_ref[...])
    h = jnp.maximum(h + b0_ref[...], 0.0)
    h = jnp.maximum(dot(h, w1_ref[...]) + b1_ref[...], 0.0)
    h = jnp.maximum(dot(h, w2_ref[...]) + b2_ref[...], 0.0)
    o = dot(h, wo_ref[...]) + bo_ref[...]
    out_ref[...] = jax.nn.sigmoid(o)


def _mlp(u, v, w0u, w0i, b0, w1t, b1, w2t, b2, wot, bo):
    grid = BATCH // _BLK
    row = lambda i: (i, 0)
    rep = lambda i: (0, 0)
    return pl.pallas_call(
        _mlp_body,
        grid=(grid,),
        in_specs=[
            pl.BlockSpec((_BLK, EMBED_DIM), row),
            pl.BlockSpec((_BLK, EMBED_DIM), row),
            pl.BlockSpec(w0u.shape, rep),
            pl.BlockSpec(w0i.shape, rep),
            pl.BlockSpec(b0.shape, rep),
            pl.BlockSpec(w1t.shape, rep),
            pl.BlockSpec(b1.shape, rep),
            pl.BlockSpec(w2t.shape, rep),
            pl.BlockSpec(b2.shape, rep),
            pl.BlockSpec(wot.shape, rep),
            pl.BlockSpec(bo.shape, rep),
        ],
        out_specs=pl.BlockSpec((_BLK, 1), row),
        out_shape=jax.ShapeDtypeStruct((BATCH, 1), jnp.float32),
    )(u, v, w0u, w0i, b0, w1t, b1, w2t, b2, wot, bo)


def kernel(user_ids, item_ids, user_table, item_table,
           W0, b0, W1, b1, W2, b2, Wo, bo):
    u_emb = _make_gather_stream(BATCH, EMBED_DIM)(
        user_table, user_ids.astype(jnp.int32))
    i_emb = _make_gather_dma(BATCH, EMBED_DIM)(
        item_table, item_ids.astype(jnp.int32))
    w0u = W0[:, :EMBED_DIM].T
    w0i = W0[:, EMBED_DIM:].T
    return _mlp(u_emb, i_emb, w0u, w0i, b0.reshape(1, -1),
                W1.T, b1.reshape(1, -1), W2.T, b2.reshape(1, -1),
                Wo.T, bo.reshape(1, 1))


# speedup vs baseline: 2.7883x; 1.3545x over previous
import functools

import jax
import jax.numpy as jnp
from jax import lax
from jax.experimental import pallas as pl
from jax.experimental.pallas import tpu as pltpu
from jax.experimental.pallas import tpu_sc as plsc

BATCH = 16384
EMBED_DIM = 64


@functools.lru_cache(maxsize=None)
def _make_gather(batch, dim):
    info = plsc.get_sparse_core_info()
    nc, ns = info.num_cores, info.num_subcores
    nw = nc * ns
    assert batch % (8 * nw) == 0
    bpw = batch // nw
    mesh = plsc.VectorSubcoreMesh(core_axis_name="c", subcore_axis_name="s")

    @functools.partial(
        pl.kernel,
        mesh=mesh,
        out_type=(
            jax.ShapeDtypeStruct((batch, dim), jnp.float32),
            jax.ShapeDtypeStruct((batch, dim), jnp.float32),
        ),
        scratch_types=[
            pltpu.VMEM((bpw,), jnp.int32),
            pltpu.VMEM((bpw,), jnp.int32),
            pltpu.VMEM((bpw, dim), jnp.float32),
            pltpu.SemaphoreType.DMA,
        ],
    )
    def gather2(ut_hbm, uid_hbm, it_hbm, iid_hbm, uo_hbm, io_hbm,
                uidx_v, iidx_v, rows_v, sem):
        wid = lax.axis_index("s") * nc + lax.axis_index("c")
        base = wid * bpw
        pltpu.sync_copy(uid_hbm.at[pl.ds(base, bpw)], uidx_v)
        pltpu.sync_copy(iid_hbm.at[pl.ds(base, bpw)], iidx_v)

        def one_table(tab, idx_v, out_hbm):
            def fire(g, _):
                vec = idx_v[pl.ds(g * 16, 16)]
                for lane in range(16):
                    rid = lax.squeeze(
                        lax.slice(vec, (lane,), (lane + 1,)), (0,))
                    pltpu.async_copy(tab.at[pl.ds(rid, 1)],
                                     rows_v.at[pl.ds(g * 16 + lane, 1)], sem)
                return 0

            lax.fori_loop(0, bpw // 16, fire, 0)

            def drain(i, _):
                pltpu.make_async_copy(tab.at[pl.ds(0, 1)],
                                      rows_v.at[pl.ds(0, 1)], sem).wait()
                return 0

            lax.fori_loop(0, bpw, drain, 0)
            pltpu.sync_copy(rows_v, out_hbm.at[pl.ds(base, bpw)])

        one_table(ut_hbm, uidx_v, uo_hbm)
        one_table(it_hbm, iidx_v, io_hbm)

    return gather2


_BLK = 2048


def _mlp_body(u_ref, v_ref, w0u_ref, w0i_ref, b0_ref, w1_ref, b1_ref,
              w2_ref, b2_ref, wo_ref, bo_ref, out_ref):
    dot = functools.partial(jnp.dot, preferred_element_type=jnp.float32)
    h = dot(u_ref[...], w0u_ref[...]) + dot(v_ref[...], w0i_ref[...])
    h = jnp.maximum(h + b0_ref[...], 0.0)
    h = jnp.maximum(dot(h, w1_ref[...]) + b1_ref[...], 0.0)
    h = jnp.maximum(dot(h, w2_ref[...]) + b2_ref[...], 0.0)
    o = dot(h, wo_ref[...]) + bo_ref[...]
    out_ref[...] = jax.nn.sigmoid(o)


def _mlp(u, v, w0u, w0i, b0, w1t, b1, w2t, b2, wot, bo):
    grid = BATCH // _BLK
    row = lambda i: (i, 0)
    rep = lambda i: (0, 0)
    return pl.pallas_call(
        _mlp_body,
        grid=(grid,),
        in_specs=[
            pl.BlockSpec((_BLK, EMBED_DIM), row),
            pl.BlockSpec((_BLK, EMBED_DIM), row),
            pl.BlockSpec(w0u.shape, rep),
            pl.BlockSpec(w0i.shape, rep),
            pl.BlockSpec(b0.shape, rep),
            pl.BlockSpec(w1t.shape, rep),
            pl.BlockSpec(b1.shape, rep),
            pl.BlockSpec(w2t.shape, rep),
            pl.BlockSpec(b2.shape, rep),
            pl.BlockSpec(wot.shape, rep),
            pl.BlockSpec(bo.shape, rep),
        ],
        out_specs=pl.BlockSpec((_BLK, 1), row),
        out_shape=jax.ShapeDtypeStruct((BATCH, 1), jnp.float32),
    )(u, v, w0u, w0i, b0, w1t, b1, w2t, b2, wot, bo)


def kernel(user_ids, item_ids, user_table, item_table,
           W0, b0, W1, b1, W2, b2, Wo, bo):
    u_emb, i_emb = _make_gather(BATCH, EMBED_DIM)(
        user_table, user_ids.astype(jnp.int32),
        item_table, item_ids.astype(jnp.int32))
    w0u = W0[:, :EMBED_DIM].T
    w0i = W0[:, EMBED_DIM:].T
    return _mlp(u_emb, i_emb, w0u, w0i, b0.reshape(1, -1),
                W1.T, b1.reshape(1, -1), W2.T, b2.reshape(1, -1),
                Wo.T, bo.reshape(1, 1))
